# baseline (speedup 1.0000x reference)
import functools

import jax
import jax.numpy as jnp
import numpy as np
from jax.experimental import pallas as pl
from jax.experimental.pallas import tpu as pltpu

N_Q_ = 8
DIM_ = 128
K_ = 1024
B_ = 16
T_ = 2048
BT_ = B_ * T_
TILE = 1024


def _rvq_body(x_ref, cb_ref, cbn_ref, q_ref, codes_ref, loss_ref):
    i = pl.program_id(0)

    @pl.when(i == 0)
    def _init():
        for q in range(N_Q_):
            loss_ref[q, 0] = 0.0

    r = x_ref[...]
    acc = jnp.zeros_like(r)
    for q in range(N_Q_):
        cb = cb_ref[q]
        rn = jnp.sum(r * r, axis=1, keepdims=True)
        xc = jax.lax.dot_general(
            r.astype(jnp.bfloat16), cb.astype(jnp.bfloat16),
            (((1,), (1,)), ((), ())),
            preferred_element_type=jnp.float32)
        dist = -(rn - 2.0 * xc + cbn_ref[q][None, :])
        ind = jnp.argmax(dist, axis=1)
        onehot = (jax.lax.broadcasted_iota(jnp.int32, (TILE, K_), 1)
                  == ind[:, None]).astype(jnp.float32)
        quant = jax.lax.dot_general(
            onehot, cb, (((1,), (0,)), ((), ())),
            preferred_element_type=jnp.float32,
            precision=jax.lax.Precision.HIGHEST)
        codes_ref[q, :] = ind
        loss_ref[q, 0] += jnp.sum((quant - r) ** 2)
        r = r - quant
        acc = acc + quant
    q_ref[...] = acc


@jax.jit
def _rvq(x, embed, frame_rate):
    xt = jnp.transpose(x, (0, 2, 1)).reshape(BT_, DIM_)
    cb_norms = jnp.sum(embed * embed, axis=-1)

    grid = (BT_ // TILE,)
    quantized_flat, codes_flat, loss_sums = pl.pallas_call(
        _rvq_body,
        grid=grid,
        in_specs=[
            pl.BlockSpec((TILE, DIM_), lambda i: (i, 0)),
            pl.BlockSpec((N_Q_, K_, DIM_), lambda i: (0, 0, 0)),
            pl.BlockSpec((N_Q_, K_), lambda i: (0, 0)),
        ],
        out_specs=[
            pl.BlockSpec((TILE, DIM_), lambda i: (i, 0)),
            pl.BlockSpec((N_Q_, TILE), lambda i: (0, i)),
            pl.BlockSpec(memory_space=pltpu.SMEM),
        ],
        out_shape=[
            jax.ShapeDtypeStruct((BT_, DIM_), jnp.float32),
            jax.ShapeDtypeStruct((N_Q_, BT_), jnp.int32),
            jax.ShapeDtypeStruct((N_Q_, 1), jnp.float32),
        ],
    )(xt, embed, cb_norms)

    quantized = jnp.transpose(
        quantized_flat.reshape(B_, T_, DIM_), (0, 2, 1))
    codes = codes_flat.reshape(N_Q_, B_, T_)
    losses = loss_sums[:, 0] / jnp.float32(BT_ * DIM_)
    penalty = jnp.mean(losses)
    bandwidth = (jnp.float32(N_Q_ * np.log2(K_))
                 * jnp.asarray(frame_rate, jnp.float32))
    return quantized, codes, bandwidth, penalty


def kernel(x, embed, frame_rate):
    return _rvq(x, embed, frame_rate)


# speedup vs baseline: 4.0824x; 4.0824x over previous
import jax
import jax.numpy as jnp
import numpy as np
from jax.experimental import pallas as pl
from jax.experimental.pallas import tpu as pltpu

N_Q_ = 8
DIM_ = 128
K_ = 1024
B_ = 16
T_ = 2048
BT_ = B_ * T_
TILE = 1024
KAUG = DIM_ + 8


def _rvq_body(x_ref, aug_ref, cbs_ref, q_ref, codes_ref, loss_ref):
    i = pl.program_id(0)

    @pl.when(i == 0)
    def _init():
        for q in range(N_Q_):
            loss_ref[q, 0] = 0.0

    r = x_ref[...]
    acc = jnp.zeros_like(r)
    iota = jax.lax.broadcasted_iota(jnp.int32, (TILE, K_), 1)
    ones_pad = jnp.concatenate(
        [jnp.ones((TILE, 3), jnp.bfloat16),
         jnp.zeros((TILE, 5), jnp.bfloat16)], axis=1)
    for q in range(N_Q_):
        ra = jnp.concatenate([r.astype(jnp.bfloat16), ones_pad], axis=1)
        score = jax.lax.dot_general(
            ra, aug_ref[q], (((1,), (1,)), ((), ())),
            preferred_element_type=jnp.float32)
        ind = jnp.argmax(score, axis=1).astype(jnp.int32)
        oh = jnp.where(iota == ind[:, None],
                       jnp.float32(1), jnp.float32(0)).astype(jnp.bfloat16)
        qs = jax.lax.dot_general(
            oh, cbs_ref[q], (((1,), (0,)), ((), ())),
            preferred_element_type=jnp.float32)
        qh = qs[:, 0:DIM_]
        qm = qs[:, DIM_:2 * DIM_]
        ql = qs[:, 2 * DIM_:]
        codes_ref[q, :] = ind
        r = ((r - qh) - qm) - ql
        acc = ((acc + qh) + qm) + ql
        loss_ref[q, 0] += jnp.sum(r * r)
    q_ref[...] = acc


@jax.jit
def _rvq(x, embed, frame_rate):
    xt = jnp.transpose(x, (0, 2, 1)).reshape(BT_, DIM_)

    hi = jax.lax.optimization_barrier(embed.astype(jnp.bfloat16))
    r1 = embed - hi.astype(jnp.float32)
    mid = jax.lax.optimization_barrier(r1.astype(jnp.bfloat16))
    r2 = r1 - mid.astype(jnp.float32)
    lo = r2.astype(jnp.bfloat16)
    cbs = jnp.concatenate([hi, mid, lo], axis=-1)

    cbn = jnp.sum(embed * embed, axis=-1)
    nhi = jax.lax.optimization_barrier(cbn.astype(jnp.bfloat16))
    s1 = cbn - nhi.astype(jnp.float32)
    nmid = jax.lax.optimization_barrier(s1.astype(jnp.bfloat16))
    s2 = s1 - nmid.astype(jnp.float32)
    nlo = s2.astype(jnp.bfloat16)
    aug = jnp.concatenate(
        [(hi.astype(jnp.float32) * 2.0).astype(jnp.bfloat16),
         -nhi[..., None], -nmid[..., None], -nlo[..., None],
         jnp.zeros((N_Q_, K_, 5), jnp.bfloat16)], axis=-1)

    grid = (BT_ // TILE,)
    quantized_flat, codes_flat, loss_sums = pl.pallas_call(
        _rvq_body,
        grid=grid,
        in_specs=[
            pl.BlockSpec((TILE, DIM_), lambda i: (i, 0)),
            pl.BlockSpec((N_Q_, K_, KAUG), lambda i: (0, 0, 0)),
            pl.BlockSpec((N_Q_, K_, 3 * DIM_), lambda i: (0, 0, 0)),
        ],
        out_specs=[
            pl.BlockSpec((TILE, DIM_), lambda i: (i, 0)),
            pl.BlockSpec((N_Q_, TILE), lambda i: (0, i)),
            pl.BlockSpec(memory_space=pltpu.SMEM),
        ],
        out_shape=[
            jax.ShapeDtypeStruct((BT_, DIM_), jnp.float32),
            jax.ShapeDtypeStruct((N_Q_, BT_), jnp.int32),
            jax.ShapeDtypeStruct((N_Q_, 1), jnp.float32),
        ],
    )(xt, aug, cbs)

    quantized = jnp.transpose(
        quantized_flat.reshape(B_, T_, DIM_), (0, 2, 1))
    codes = codes_flat.reshape(N_Q_, B_, T_)
    losses = loss_sums[:, 0] / jnp.float32(BT_ * DIM_)
    penalty = jnp.mean(losses)
    bandwidth = (jnp.float32(N_Q_ * np.log2(K_))
                 * jnp.asarray(frame_rate, jnp.float32))
    return quantized, codes, bandwidth, penalty


def kernel(x, embed, frame_rate):
    return _rvq(x, embed, frame_rate)
